# Initial kernel scaffold; baseline (speedup 1.0000x reference)
import jax
import jax.numpy as jnp
from jax.experimental import pallas as pl


def kernel(x, table, bw0, bb0, bw1, bb1, bw2, bb2, tw0, tb0, tw1, tb1, tw2, tb2, tw3, tb3, tw4, tb4):
    raise NotImplementedError("write your pallas kernel here")


# speedup vs baseline: 3.8683x; 3.8683x over previous
import jax
import jax.numpy as jnp
import numpy as np
from jax.experimental import pallas as pl
from jax.experimental.pallas import tpu as pltpu
from jax.experimental.pallas import tpu_sc as plsc

VOCAB = 1000000
EMBED = 128
NDENSE = 13
NSPARSE = 26
NFEAT = NSPARSE + 1
BB = 256
GW = 128


def _sc_gather(table, idx):
    n = idx.shape[1]
    mesh = plsc.VectorSubcoreMesh(core_axis_name="core",
                                  subcore_axis_name="subcore")

    @pl.kernel(out_type=jax.ShapeDtypeStruct((n, EMBED), table.dtype),
               mesh=mesh)
    def run(tab_hbm, idx_hbm, out_hbm):
        def body(i_vmem, o_vmem):
            pltpu.sync_copy(tab_hbm.at[i_vmem.at[0]], o_vmem)

        pltpu.emit_pipeline(
            body,
            grid=(n // GW,),
            in_specs=[pl.BlockSpec((1, GW), lambda i: (0, i))],
            out_specs=[pl.BlockSpec((GW, EMBED), lambda i: (i, 0))],
            core_axis_name=("core", "subcore"),
            dimension_semantics=(pltpu.PARALLEL,),
        )(idx_hbm, out_hbm)

    return run(table, idx)


def _tc_body(x_ref, emb_ref, bw0, bb0, bw1, bb1, bw2, bb2,
             a0, w2, tb0, tw1, tb1, tw2, tb2, tw3, tb3, tw4, tb4,
             o_ref, xp_ref):
    f32 = jnp.float32

    @pl.when(pl.program_id(0) == 0)
    def _():
        xp_ref[...] = jnp.zeros_like(xp_ref)

    h = x_ref[:, :NDENSE]
    h = jax.nn.relu(jnp.dot(h, bw0[...], preferred_element_type=f32) + bb0[...])
    h = jax.nn.relu(jnp.dot(h, bw1[...], preferred_element_type=f32) + bb1[...])
    bot = jax.nn.relu(jnp.dot(h, bw2[...], preferred_element_type=f32) + bb2[...])

    emb = emb_ref[...].reshape(BB, NSPARSE, EMBED)
    s = jnp.concatenate([bot[:, None, :], emb], axis=1)
    for nf in range(NFEAT):
        row = jnp.sum(s[:, nf:, :] * s[:, nf:nf + 1, :], axis=-1)
        xp_ref[:, nf * EMBED: nf * EMBED + (NFEAT - nf)] = row

    h = jnp.dot(bot, a0[...], preferred_element_type=f32)
    h = h + jnp.dot(xp_ref[...], w2[...], preferred_element_type=f32)
    h = jax.nn.relu(h + tb0[...])
    h = jax.nn.relu(jnp.dot(h, tw1[...], preferred_element_type=f32) + tb1[...])
    h = jax.nn.relu(jnp.dot(h, tw2[...], preferred_element_type=f32) + tb2[...])
    h = jax.nn.relu(jnp.dot(h, tw3[...], preferred_element_type=f32) + tb3[...])
    o_ref[...] = jnp.dot(h, tw4[...], preferred_element_type=f32) + tb4[...]


def _full(arr):
    return pl.BlockSpec(arr.shape, lambda i: (0,) * arr.ndim)


def kernel(x, table, bw0, bb0, bw1, bb1, bw2, bb2,
           tw0, tb0, tw1, tb1, tw2, tb2, tw3, tb3, tw4, tb4):
    batch = x.shape[0]
    idx = (x[:, NDENSE:].astype(jnp.int32) % VOCAB).reshape(1, batch * NSPARSE)
    emb = _sc_gather(table, idx)

    iu, ju = np.triu_indices(NFEAT)
    rows = iu * EMBED + (ju - iu)
    w2 = jnp.zeros((NFEAT * EMBED, tw0.shape[1]), tw0.dtype).at[rows].set(tw0[EMBED:])
    a0 = tw0[:EMBED]

    weights = (bw0, bb0.reshape(1, -1), bw1, bb1.reshape(1, -1),
               bw2, bb2.reshape(1, -1), a0, w2, tb0.reshape(1, -1),
               tw1, tb1.reshape(1, -1), tw2, tb2.reshape(1, -1),
               tw3, tb3.reshape(1, -1), tw4, tb4.reshape(1, -1))

    out = pl.pallas_call(
        _tc_body,
        grid=(batch // BB,),
        in_specs=[
            pl.BlockSpec((BB, x.shape[1]), lambda i: (i, 0)),
            pl.BlockSpec((BB * NSPARSE, EMBED), lambda i: (i, 0)),
        ] + [_full(w) for w in weights],
        out_specs=pl.BlockSpec((BB, 1), lambda i: (i, 0)),
        out_shape=jax.ShapeDtypeStruct((batch, 1), jnp.float32),
        scratch_shapes=[pltpu.VMEM((BB, NFEAT * EMBED), jnp.float32)],
    )(x, emb, *weights)
    return out
